# baseline (speedup 1.0000x reference)
import functools

import jax
import jax.numpy as jnp
from jax import lax
from jax.experimental import pallas as pl
from jax.experimental.pallas import tpu as pltpu
from jax.experimental.pallas import tpu_sc as plsc

B, N, D = 16, 1024, 256
K = 8192
T = 256
G = (B * N) // T
KS = 2048

NC, NS = 2, 16
NW = NC * NS
ROWS_PER_W = (B * N) // NW
CHUNK = 128
NCHUNK = ROWS_PER_W // CHUNK


def _tc_body(x_ref, cb_ref, idx_ref, loss_ref, csq_ref, kidx_ref):
    i = pl.program_id(0)

    @pl.when(i == 0)
    def _():
        loss_ref[0, 0] = 0.0
        csq_ref[0, :] = jnp.sum(cb_ref[...] * cb_ref[...], axis=1)
        kidx_ref[0, :] = lax.broadcasted_iota(jnp.int32, (K,), 0)

    xb = x_ref[...]
    xb2 = xb + xb
    xsq = jnp.sum(xb * xb, axis=1, keepdims=True)
    bestv = jnp.full((T, 1), jnp.inf, jnp.float32)
    bestj = jnp.zeros((T, 1), jnp.int32)
    for j in range(K // KS):
        m2 = lax.dot_general(xb2, cb_ref[j * KS:(j + 1) * KS, :],
                             (((1,), (1,)), ((), ())),
                             preferred_element_type=jnp.float32)
        t = (xsq - m2) + csq_ref[0, j * KS:(j + 1) * KS][None, :]
        mc = jnp.min(t, axis=1, keepdims=True)
        cnd = jnp.where(t == mc, kidx_ref[0, j * KS:(j + 1) * KS][None, :], K)
        ic = jnp.min(cnd, axis=1, keepdims=True)
        upd = mc < bestv
        bestj = jnp.where(upd, ic, bestj)
        bestv = jnp.where(upd, mc, bestv)
    idx_ref[0, 0, :] = bestj.reshape(T)
    loss_ref[0, 0] += jnp.sum(bestv) * (0.25 / (B * N * D))


def _tc_argmax(flat_x, codebook):
    rows = flat_x.shape[0]
    g = rows // T
    return pl.pallas_call(
        _tc_body,
        grid=(g,),
        in_specs=[
            pl.BlockSpec((T, D), lambda i: (i, 0)),
            pl.BlockSpec((K, D), lambda i: (0, 0)),
        ],
        out_specs=[
            pl.BlockSpec((1, 1, T), lambda i: (i, 0, 0)),
            pl.BlockSpec(memory_space=pltpu.SMEM, block_shape=(1, 1),
                         index_map=lambda i: (0, 0)),
        ],
        out_shape=[
            jax.ShapeDtypeStruct((g, 1, T), jnp.int32),
            jax.ShapeDtypeStruct((1, 1), jnp.float32),
        ],
        scratch_shapes=[pltpu.VMEM((1, K), jnp.float32),
                        pltpu.VMEM((1, K), jnp.int32)],
    )(flat_x, codebook)


@functools.cache
def _build_sc_gather(rows):
    rows_per_w = rows // NW
    nchunk = rows_per_w // CHUNK

    @functools.partial(
        pl.kernel,
        mesh=plsc.VectorSubcoreMesh(core_axis_name="c", subcore_axis_name="s"),
        out_type=jax.ShapeDtypeStruct((rows, D), jnp.float32),
        scratch_types=[
            pltpu.VMEM((rows_per_w,), jnp.int32),
            pltpu.VMEM((CHUNK, D), jnp.float32),
            pltpu.SemaphoreType.DMA,
        ],
    )
    def _sc_gather(cb_hbm, idx_hbm, out_hbm, idx_v, rows_v, sem):
        wid = lax.axis_index("s") * NC + lax.axis_index("c")
        base = wid * rows_per_w
        pltpu.sync_copy(idx_hbm.at[pl.ds(base, rows_per_w)], idx_v)
        for c in range(nchunk):
            pltpu.async_copy(cb_hbm.at[idx_v.at[pl.ds(c * CHUNK, CHUNK)]],
                             rows_v, sem).wait()
            pltpu.sync_copy(rows_v, out_hbm.at[pl.ds(base + c * CHUNK, CHUNK)])

    return _sc_gather


NSLICE = 2
HALF = (B * N) // NSLICE


def kernel(x, codebook):
    flat_x = x.reshape(B * N, D)
    gather = _build_sc_gather(HALF)
    idxs, losses, quants = [], [], []
    for s in range(NSLICE):
        idx3, loss = _tc_argmax(flat_x[s * HALF:(s + 1) * HALF], codebook)
        idxs.append(idx3)
        losses.append(loss)
        quants.append(gather(codebook, idx3.reshape(HALF)))
    quantize_st = jnp.concatenate(quants, axis=0).reshape(B, N, D)
    embed_ind = jnp.concatenate(idxs, axis=0).reshape(B, N)
    commit_loss = sum(l[0, 0] for l in losses)
    return quantize_st, embed_ind, commit_loss


# speedup vs baseline: 1.1147x; 1.1147x over previous
import functools

import jax
import jax.numpy as jnp
from jax import lax
from jax.experimental import pallas as pl
from jax.experimental.pallas import tpu as pltpu
from jax.experimental.pallas import tpu_sc as plsc

B, N, D = 16, 1024, 256
K = 8192
T = 256
G = (B * N) // T
KS = 2048

NC, NS = 2, 16
NW = NC * NS
ROWS_PER_W = (B * N) // NW
CHUNK = 128
NCHUNK = ROWS_PER_W // CHUNK


def _tc_body(x_ref, cb_ref, idx_ref, loss_ref, csq_ref, kidx_ref):
    i = pl.program_id(0)

    @pl.when(i == 0)
    def _():
        loss_ref[0, 0] = 0.0
        csq_ref[0, :] = jnp.sum(cb_ref[...] * cb_ref[...], axis=1)
        kidx_ref[0, :] = lax.broadcasted_iota(jnp.int32, (K,), 0)

    xb = x_ref[...]
    xb2 = xb + xb
    xsq = jnp.sum(xb * xb, axis=1, keepdims=True)
    bestv = jnp.full((T, 1), jnp.inf, jnp.float32)
    bestj = jnp.zeros((T, 1), jnp.int32)
    for j in range(K // KS):
        m2 = lax.dot_general(xb2, cb_ref[j * KS:(j + 1) * KS, :],
                             (((1,), (1,)), ((), ())),
                             preferred_element_type=jnp.float32)
        t = (xsq - m2) + csq_ref[0, j * KS:(j + 1) * KS][None, :]
        mc = jnp.min(t, axis=1, keepdims=True)
        cnd = jnp.where(t == mc, kidx_ref[0, j * KS:(j + 1) * KS][None, :], K)
        ic = jnp.min(cnd, axis=1, keepdims=True)
        upd = mc < bestv
        bestj = jnp.where(upd, ic, bestj)
        bestv = jnp.where(upd, mc, bestv)
    idx_ref[0, 0, :] = bestj.reshape(T)
    loss_ref[0, 0] += jnp.sum(bestv) * (0.25 / (B * N * D))


def _tc_argmax(flat_x, codebook):
    rows = flat_x.shape[0]
    g = rows // T
    return pl.pallas_call(
        _tc_body,
        grid=(g,),
        in_specs=[
            pl.BlockSpec((T, D), lambda i: (i, 0)),
            pl.BlockSpec((K, D), lambda i: (0, 0)),
        ],
        out_specs=[
            pl.BlockSpec((1, 1, T), lambda i: (i, 0, 0)),
            pl.BlockSpec(memory_space=pltpu.SMEM, block_shape=(1, 1),
                         index_map=lambda i: (0, 0)),
        ],
        out_shape=[
            jax.ShapeDtypeStruct((g, 1, T), jnp.int32),
            jax.ShapeDtypeStruct((1, 1), jnp.float32),
        ],
        scratch_shapes=[pltpu.VMEM((1, K), jnp.float32),
                        pltpu.VMEM((1, K), jnp.int32)],
    )(flat_x, codebook)


@functools.cache
def _build_sc_gather(rows):
    rows_per_w = rows // NW
    nchunk = rows_per_w // CHUNK

    @functools.partial(
        pl.kernel,
        mesh=plsc.VectorSubcoreMesh(core_axis_name="c", subcore_axis_name="s"),
        out_type=jax.ShapeDtypeStruct((rows, D), jnp.float32),
        scratch_types=[
            pltpu.VMEM((rows_per_w,), jnp.int32),
            pltpu.VMEM((CHUNK, D), jnp.float32),
            pltpu.VMEM((CHUNK, D), jnp.float32),
            pltpu.SemaphoreType.DMA,
            pltpu.SemaphoreType.DMA,
        ],
    )
    def _sc_gather(cb_hbm, idx_hbm, out_hbm, idx_v, rows_a, rows_b,
                   gsem, wsem):
        wid = lax.axis_index("s") * NC + lax.axis_index("c")
        base = wid * rows_per_w
        pltpu.sync_copy(idx_hbm.at[pl.ds(base, rows_per_w)], idx_v)
        bufs = (rows_a, rows_b)
        gathers = []
        for c in range(nchunk):
            gathers.append(pltpu.async_copy(
                cb_hbm.at[idx_v.at[pl.ds(c * CHUNK, CHUNK)]],
                bufs[c % 2], gsem))
            if c >= 1:
                gathers[c - 1].wait()
                pltpu.async_copy(
                    bufs[(c - 1) % 2],
                    out_hbm.at[pl.ds(base + (c - 1) * CHUNK, CHUNK)],
                    wsem).wait()
        gathers[nchunk - 1].wait()
        pltpu.sync_copy(bufs[(nchunk - 1) % 2],
                        out_hbm.at[pl.ds(base + (nchunk - 1) * CHUNK, CHUNK)])

    return _sc_gather


NSLICE = 1
HALF = (B * N) // NSLICE


def kernel(x, codebook):
    flat_x = x.reshape(B * N, D)
    gather = _build_sc_gather(HALF)
    idxs, losses, quants = [], [], []
    for s in range(NSLICE):
        idx3, loss = _tc_argmax(flat_x[s * HALF:(s + 1) * HALF], codebook)
        idxs.append(idx3)
        losses.append(loss)
        quants.append(gather(codebook, idx3.reshape(HALF)))
    quantize_st = jnp.concatenate(quants, axis=0).reshape(B, N, D)
    embed_ind = jnp.concatenate(idxs, axis=0).reshape(B, N)
    commit_loss = sum(l[0, 0] for l in losses)
    return quantize_st, embed_ind, commit_loss


# speedup vs baseline: 1.2290x; 1.1026x over previous
import functools

import jax
import jax.numpy as jnp
from jax import lax
from jax.experimental import pallas as pl
from jax.experimental.pallas import tpu as pltpu
from jax.experimental.pallas import tpu_sc as plsc

B, N, D = 16, 1024, 256
K = 8192
T = 256
G = (B * N) // T
KS = 4096

NC, NS = 2, 16
NW = NC * NS
ROWS_PER_W = (B * N) // NW
CHUNK = 128
NCHUNK = ROWS_PER_W // CHUNK


def _tc_body(x_ref, cb_ref, idx_ref, loss_ref, csq_ref, kidx_ref):
    i = pl.program_id(0)

    @pl.when(i == 0)
    def _():
        loss_ref[0, 0] = 0.0
        csq_ref[0, :] = jnp.sum(cb_ref[...] * cb_ref[...], axis=1)
        kidx_ref[0, :] = lax.broadcasted_iota(jnp.int32, (K,), 0)

    xb = x_ref[...]
    xb2 = xb + xb
    xsq = jnp.sum(xb * xb, axis=1, keepdims=True)
    bestv = jnp.full((T, 1), jnp.inf, jnp.float32)
    bestj = jnp.zeros((T, 1), jnp.int32)
    for j in range(K // KS):
        m2 = lax.dot_general(xb2, cb_ref[j * KS:(j + 1) * KS, :],
                             (((1,), (1,)), ((), ())),
                             preferred_element_type=jnp.float32)
        t = (xsq - m2) + csq_ref[0, j * KS:(j + 1) * KS][None, :]
        mc = jnp.min(t, axis=1, keepdims=True)
        cnd = jnp.where(t == mc, kidx_ref[0, j * KS:(j + 1) * KS][None, :], K)
        ic = jnp.min(cnd, axis=1, keepdims=True)
        upd = mc < bestv
        bestj = jnp.where(upd, ic, bestj)
        bestv = jnp.where(upd, mc, bestv)
    idx_ref[0, 0, :] = bestj.reshape(T)
    loss_ref[0, 0] += jnp.sum(bestv) * (0.25 / (B * N * D))


def _tc_argmax(flat_x, codebook):
    rows = flat_x.shape[0]
    g = rows // T
    return pl.pallas_call(
        _tc_body,
        grid=(g,),
        in_specs=[
            pl.BlockSpec((T, D), lambda i: (i, 0)),
            pl.BlockSpec((K, D), lambda i: (0, 0)),
        ],
        out_specs=[
            pl.BlockSpec((1, 1, T), lambda i: (i, 0, 0)),
            pl.BlockSpec(memory_space=pltpu.SMEM, block_shape=(1, 1),
                         index_map=lambda i: (0, 0)),
        ],
        out_shape=[
            jax.ShapeDtypeStruct((g, 1, T), jnp.int32),
            jax.ShapeDtypeStruct((1, 1), jnp.float32),
        ],
        scratch_shapes=[pltpu.VMEM((1, K), jnp.float32),
                        pltpu.VMEM((1, K), jnp.int32)],
    )(flat_x, codebook)


@functools.cache
def _build_sc_gather(rows):
    rows_per_w = rows // NW
    nchunk = rows_per_w // CHUNK

    @functools.partial(
        pl.kernel,
        mesh=plsc.VectorSubcoreMesh(core_axis_name="c", subcore_axis_name="s"),
        out_type=jax.ShapeDtypeStruct((rows, D), jnp.float32),
        scratch_types=[
            pltpu.VMEM((rows_per_w,), jnp.int32),
            pltpu.VMEM((CHUNK, D), jnp.float32),
            pltpu.VMEM((CHUNK, D), jnp.float32),
            pltpu.SemaphoreType.DMA,
            pltpu.SemaphoreType.DMA,
        ],
    )
    def _sc_gather(cb_hbm, idx_hbm, out_hbm, idx_v, rows_a, rows_b,
                   gsem, wsem):
        wid = lax.axis_index("s") * NC + lax.axis_index("c")
        base = wid * rows_per_w
        pltpu.sync_copy(idx_hbm.at[pl.ds(base, rows_per_w)], idx_v)
        bufs = (rows_a, rows_b)
        gathers = []
        for c in range(nchunk):
            gathers.append(pltpu.async_copy(
                cb_hbm.at[idx_v.at[pl.ds(c * CHUNK, CHUNK)]],
                bufs[c % 2], gsem))
            if c >= 1:
                gathers[c - 1].wait()
                pltpu.async_copy(
                    bufs[(c - 1) % 2],
                    out_hbm.at[pl.ds(base + (c - 1) * CHUNK, CHUNK)],
                    wsem).wait()
        gathers[nchunk - 1].wait()
        pltpu.sync_copy(bufs[(nchunk - 1) % 2],
                        out_hbm.at[pl.ds(base + (nchunk - 1) * CHUNK, CHUNK)])

    return _sc_gather


NSLICE = 1
HALF = (B * N) // NSLICE


def kernel(x, codebook):
    flat_x = x.reshape(B * N, D)
    gather = _build_sc_gather(HALF)
    idxs, losses, quants = [], [], []
    for s in range(NSLICE):
        idx3, loss = _tc_argmax(flat_x[s * HALF:(s + 1) * HALF], codebook)
        idxs.append(idx3)
        losses.append(loss)
        quants.append(gather(codebook, idx3.reshape(HALF)))
    quantize_st = jnp.concatenate(quants, axis=0).reshape(B, N, D)
    embed_ind = jnp.concatenate(idxs, axis=0).reshape(B, N)
    commit_loss = sum(l[0, 0] for l in losses)
    return quantize_st, embed_ind, commit_loss


# speedup vs baseline: 1.2431x; 1.0115x over previous
import functools

import jax
import jax.numpy as jnp
from jax import lax
from jax.experimental import pallas as pl
from jax.experimental.pallas import tpu as pltpu
from jax.experimental.pallas import tpu_sc as plsc

B, N, D = 16, 1024, 256
K = 8192
T = 256
G = (B * N) // T
KS = 8192

NC, NS = 2, 16
NW = NC * NS
ROWS_PER_W = (B * N) // NW
CHUNK = 128
NCHUNK = ROWS_PER_W // CHUNK


def _tc_body(x_ref, cb_ref, idx_ref, loss_ref, csq_ref, kidx_ref):
    i = pl.program_id(0)

    @pl.when(i == 0)
    def _():
        loss_ref[0, 0] = 0.0
        csq_ref[0, :] = jnp.sum(cb_ref[...] * cb_ref[...], axis=1)
        kidx_ref[0, :] = lax.broadcasted_iota(jnp.int32, (K,), 0)

    xb = x_ref[...]
    xb2 = xb + xb
    xsq = jnp.sum(xb * xb, axis=1, keepdims=True)
    bestv = jnp.full((T, 1), jnp.inf, jnp.float32)
    bestj = jnp.zeros((T, 1), jnp.int32)
    for j in range(K // KS):
        m2 = lax.dot_general(xb2, cb_ref[j * KS:(j + 1) * KS, :],
                             (((1,), (1,)), ((), ())),
                             preferred_element_type=jnp.float32)
        t = (xsq - m2) + csq_ref[0, j * KS:(j + 1) * KS][None, :]
        mc = jnp.min(t, axis=1, keepdims=True)
        cnd = jnp.where(t == mc, kidx_ref[0, j * KS:(j + 1) * KS][None, :], K)
        ic = jnp.min(cnd, axis=1, keepdims=True)
        upd = mc < bestv
        bestj = jnp.where(upd, ic, bestj)
        bestv = jnp.where(upd, mc, bestv)
    idx_ref[0, 0, :] = bestj.reshape(T)
    loss_ref[0, 0] += jnp.sum(bestv) * (0.25 / (B * N * D))


def _tc_argmax(flat_x, codebook):
    rows = flat_x.shape[0]
    g = rows // T
    return pl.pallas_call(
        _tc_body,
        grid=(g,),
        in_specs=[
            pl.BlockSpec((T, D), lambda i: (i, 0)),
            pl.BlockSpec((K, D), lambda i: (0, 0)),
        ],
        out_specs=[
            pl.BlockSpec((1, 1, T), lambda i: (i, 0, 0)),
            pl.BlockSpec(memory_space=pltpu.SMEM, block_shape=(1, 1),
                         index_map=lambda i: (0, 0)),
        ],
        out_shape=[
            jax.ShapeDtypeStruct((g, 1, T), jnp.int32),
            jax.ShapeDtypeStruct((1, 1), jnp.float32),
        ],
        scratch_shapes=[pltpu.VMEM((1, K), jnp.float32),
                        pltpu.VMEM((1, K), jnp.int32)],
    )(flat_x, codebook)


@functools.cache
def _build_sc_gather(rows):
    rows_per_w = rows // NW
    nchunk = rows_per_w // CHUNK

    @functools.partial(
        pl.kernel,
        mesh=plsc.VectorSubcoreMesh(core_axis_name="c", subcore_axis_name="s"),
        out_type=jax.ShapeDtypeStruct((rows, D), jnp.float32),
        scratch_types=[
            pltpu.VMEM((rows_per_w,), jnp.int32),
            pltpu.VMEM((CHUNK, D), jnp.float32),
            pltpu.VMEM((CHUNK, D), jnp.float32),
            pltpu.SemaphoreType.DMA,
            pltpu.SemaphoreType.DMA,
        ],
    )
    def _sc_gather(cb_hbm, idx_hbm, out_hbm, idx_v, rows_a, rows_b,
                   gsem, wsem):
        wid = lax.axis_index("s") * NC + lax.axis_index("c")
        base = wid * rows_per_w
        pltpu.sync_copy(idx_hbm.at[pl.ds(base, rows_per_w)], idx_v)
        bufs = (rows_a, rows_b)
        gathers = []
        for c in range(nchunk):
            gathers.append(pltpu.async_copy(
                cb_hbm.at[idx_v.at[pl.ds(c * CHUNK, CHUNK)]],
                bufs[c % 2], gsem))
            if c >= 1:
                gathers[c - 1].wait()
                pltpu.async_copy(
                    bufs[(c - 1) % 2],
                    out_hbm.at[pl.ds(base + (c - 1) * CHUNK, CHUNK)],
                    wsem).wait()
        gathers[nchunk - 1].wait()
        pltpu.sync_copy(bufs[(nchunk - 1) % 2],
                        out_hbm.at[pl.ds(base + (nchunk - 1) * CHUNK, CHUNK)])

    return _sc_gather


NSLICE = 1
HALF = (B * N) // NSLICE


def kernel(x, codebook):
    flat_x = x.reshape(B * N, D)
    gather = _build_sc_gather(HALF)
    idxs, losses, quants = [], [], []
    for s in range(NSLICE):
        idx3, loss = _tc_argmax(flat_x[s * HALF:(s + 1) * HALF], codebook)
        idxs.append(idx3)
        losses.append(loss)
        quants.append(gather(codebook, idx3.reshape(HALF)))
    quantize_st = jnp.concatenate(quants, axis=0).reshape(B, N, D)
    embed_ind = jnp.concatenate(idxs, axis=0).reshape(B, N)
    commit_loss = sum(l[0, 0] for l in losses)
    return quantize_st, embed_ind, commit_loss
